# baseline (speedup 1.0000x reference)
import functools

import jax
import jax.numpy as jnp
from jax import lax
from jax.experimental import pallas as pl
from jax.experimental.pallas import tpu as pltpu
from jax.experimental.pallas import tpu_sc as plsc

NW = 32
CH = 4
LANES = 16


def _make_sc_kernel(B, T, D):
    TW = T // NW
    NCHUNK = TW // CH
    CHW = CH * D

    mesh = plsc.VectorSubcoreMesh(core_axis_name="c", subcore_axis_name="s")

    @functools.partial(
        pl.kernel,
        out_type=jax.ShapeDtypeStruct((B * T * D,), jnp.float32),
        mesh=mesh,
        scratch_types=[
            pltpu.VMEM((2, CHW), jnp.float32),
            pltpu.VMEM((2, B, CHW), jnp.float32),
            pltpu.SemaphoreType.DMA((2,)),
            pltpu.SemaphoreType.DMA((2,)),
            pltpu.SemaphoreType.DMA((2,)),
        ],
    )
    def sc_add(x_hbm, wpe_hbm, out_hbm, wpe_buf, x_buf, wpe_sem, x_sem, out_sem):
        wid = lax.axis_index("s") * 2 + lax.axis_index("c")
        t0w = wid * TW

        def xword(b, t0):
            return (b * T + t0) * D

        def start_in(c, p):
            t0 = t0w + c * CH
            pltpu.async_copy(
                wpe_hbm.at[pl.ds(t0 * D, CHW)], wpe_buf.at[p], wpe_sem.at[p])
            for b in range(B):
                pltpu.async_copy(
                    x_hbm.at[pl.ds(xword(b, t0), CHW)], x_buf.at[p, b],
                    x_sem.at[p])

        def wait_in(p):
            pltpu.make_async_copy(
                wpe_hbm.at[pl.ds(0, CHW)], wpe_buf.at[p], wpe_sem.at[p]).wait()
            for b in range(B):
                pltpu.make_async_copy(
                    x_hbm.at[pl.ds(0, CHW)], x_buf.at[p, b], x_sem.at[p]).wait()

        def add_chunk(p):
            for b in range(B):
                @plsc.parallel_loop(0, CHW // LANES, unroll=8)
                def _(i):
                    sl = pl.ds(i * LANES, LANES)
                    plsc.addupdate(x_buf.at[p, b, sl], wpe_buf[p, sl])

        def start_out(c, p):
            t0 = t0w + c * CH
            for b in range(B):
                pltpu.async_copy(
                    x_buf.at[p, b], out_hbm.at[pl.ds(xword(b, t0), CHW)],
                    out_sem.at[p])

        def wait_out(p):
            for b in range(B):
                pltpu.make_async_copy(
                    x_buf.at[p, b], out_hbm.at[pl.ds(0, CHW)],
                    out_sem.at[p]).wait()

        start_in(0, 0)
        wait_in(0)
        start_in(1, 1)
        add_chunk(0)
        start_out(0, 0)

        def pair(j, carry):
            for k, p in ((0, 1), (1, 0)):
                c = 2 * j + 1 + k
                wait_in(p)
                wait_out(1 - p)
                start_in(c + 1, 1 - p)
                add_chunk(p)
                start_out(c, p)
            return carry

        lax.fori_loop(0, (NCHUNK - 2) // 2, pair, 0)

        wait_in(1)
        wait_out(0)
        add_chunk(1)
        start_out(NCHUNK - 1, 1)
        wait_out(1)

    return sc_add


def kernel(x, wpe):
    b, t, d = x.shape
    sc_add = _make_sc_kernel(b, t, d)
    out = sc_add(x.reshape(-1), wpe.reshape(-1))
    return out.reshape(b, t, d)


# speedup vs baseline: 3.5647x; 3.5647x over previous
import functools

import jax
import jax.numpy as jnp
from jax import lax
from jax.experimental import pallas as pl
from jax.experimental.pallas import tpu as pltpu
from jax.experimental.pallas import tpu_sc as plsc

NW = 32
CH = 8
LANES = 16


def _make_sc_kernel(B, T, D):
    TW = T // NW
    NCHUNK = TW // CH

    mesh = plsc.VectorSubcoreMesh(core_axis_name="c", subcore_axis_name="s")

    @functools.partial(
        pl.kernel,
        out_type=jax.ShapeDtypeStruct((B, T, D), jnp.float32),
        mesh=mesh,
        compiler_params=pltpu.CompilerParams(use_tc_tiling_on_sc=True),
        scratch_types=[
            pltpu.VMEM((2, CH, D), jnp.float32),
            pltpu.VMEM((B, CH, D), jnp.float32),
            pltpu.SemaphoreType.DMA((2,)),
            pltpu.SemaphoreType.DMA((B,)),
            pltpu.SemaphoreType.DMA((B,)),
        ],
    )
    def sc_add(x_hbm, wpe_hbm, out_hbm, wpe_buf, x_buf, wpe_sem, x_sem, out_sem):
        wid = lax.axis_index("s") * 2 + lax.axis_index("c")
        t0w = wid * TW

        def start_x(c, b):
            t0 = t0w + c * CH
            pltpu.async_copy(
                x_hbm.at[b, pl.ds(t0, CH)], x_buf.at[b], x_sem.at[b])

        def wait_x(b):
            pltpu.make_async_copy(
                x_hbm.at[b, pl.ds(0, CH)], x_buf.at[b], x_sem.at[b]).wait()

        def start_wpe(c, p):
            t0 = t0w + c * CH
            pltpu.async_copy(
                wpe_hbm.at[pl.ds(t0, CH)], wpe_buf.at[p], wpe_sem.at[p])

        def wait_wpe(p):
            pltpu.make_async_copy(
                wpe_hbm.at[pl.ds(0, CH)], wpe_buf.at[p], wpe_sem.at[p]).wait()

        def start_out(c, b):
            t0 = t0w + c * CH
            pltpu.async_copy(
                x_buf.at[b], out_hbm.at[b, pl.ds(t0, CH)], out_sem.at[b])

        def wait_out(b):
            pltpu.make_async_copy(
                x_buf.at[b], out_hbm.at[b, pl.ds(0, CH)], out_sem.at[b]).wait()

        def add_step(b, p):
            for r in range(CH):
                @plsc.parallel_loop(0, D // LANES, unroll=8)
                def _(i):
                    sl = pl.ds(i * LANES, LANES)
                    plsc.addupdate(x_buf.at[b, r, sl], wpe_buf[p, r, sl])

        def step(c, b, p, *, first=False, last=False):
            if not first:
                wait_out((b + 2) % B)
            if b < 2:
                start_x(c, b + 2)
            elif not last:
                start_x(c + 1, b - 2)
            wait_x(b)
            if b == 0:
                wait_wpe(p)
                if not last:
                    start_wpe(c + 1, 1 - p)
            add_step(b, p)
            start_out(c, b)

        start_wpe(0, 0)
        start_x(0, 0)
        start_x(0, 1)
        step(0, 0, 0, first=True)
        step(0, 1, 0, first=True)
        step(0, 2, 0)
        step(0, 3, 0)

        def pair(j, carry):
            c = 2 * j + 1
            for b in range(B):
                step(c, b, 1)
            for b in range(B):
                step(c + 1, b, 0)
            return carry

        lax.fori_loop(0, (NCHUNK - 2) // 2, pair, 0)

        cl = NCHUNK - 1
        for b in range(B):
            step(cl, b, 1, last=True)
        wait_out(2)
        wait_out(3)

    return sc_add


def kernel(x, wpe):
    b, t, d = x.shape
    sc_add = _make_sc_kernel(b, t, d)
    return sc_add(x, wpe)


# speedup vs baseline: 3.5687x; 1.0011x over previous
import functools

import jax
import jax.numpy as jnp
from jax import lax
from jax.experimental import pallas as pl
from jax.experimental.pallas import tpu as pltpu
from jax.experimental.pallas import tpu_sc as plsc

NW = 32
CH = 8
LANES = 16


def _make_sc_kernel(B, T, D):
    TW = T // NW
    NCHUNK = TW // CH

    mesh = plsc.VectorSubcoreMesh(core_axis_name="c", subcore_axis_name="s")

    @functools.partial(
        pl.kernel,
        out_type=jax.ShapeDtypeStruct((B, T, D), jnp.float32),
        mesh=mesh,
        compiler_params=pltpu.CompilerParams(
            use_tc_tiling_on_sc=True,
            disable_bounds_checks=True,
        ),
        scratch_types=[
            pltpu.VMEM((2, CH, D), jnp.float32),
            pltpu.VMEM((B, CH, D), jnp.float32),
            pltpu.SemaphoreType.DMA((2,)),
            pltpu.SemaphoreType.DMA((B,)),
            pltpu.SemaphoreType.DMA((B,)),
        ],
    )
    def sc_add(x_hbm, wpe_hbm, out_hbm, wpe_buf, x_buf, wpe_sem, x_sem, out_sem):
        wid = lax.axis_index("s") * 2 + lax.axis_index("c")
        t0w = wid * TW

        def start_x(c, b):
            t0 = t0w + c * CH
            pltpu.async_copy(
                x_hbm.at[b, pl.ds(t0, CH)], x_buf.at[b], x_sem.at[b])

        def wait_x(b):
            pltpu.make_async_copy(
                x_hbm.at[b, pl.ds(0, CH)], x_buf.at[b], x_sem.at[b]).wait()

        def start_wpe(c, p):
            t0 = t0w + c * CH
            pltpu.async_copy(
                wpe_hbm.at[pl.ds(t0, CH)], wpe_buf.at[p], wpe_sem.at[p])

        def wait_wpe(p):
            pltpu.make_async_copy(
                wpe_hbm.at[pl.ds(0, CH)], wpe_buf.at[p], wpe_sem.at[p]).wait()

        def start_out(c, b):
            t0 = t0w + c * CH
            pltpu.async_copy(
                x_buf.at[b], out_hbm.at[b, pl.ds(t0, CH)], out_sem.at[b])

        def wait_out(b):
            pltpu.make_async_copy(
                x_buf.at[b], out_hbm.at[b, pl.ds(0, CH)], out_sem.at[b]).wait()

        def add_step(b, p):
            for r in range(CH):
                @plsc.parallel_loop(0, D // LANES, unroll=8)
                def _(i):
                    sl = pl.ds(i * LANES, LANES)
                    plsc.addupdate(x_buf.at[b, r, sl], wpe_buf[p, r, sl])

        def step(c, b, p, *, first=False, last=False):
            if not first:
                wait_out((b + 2) % B)
            if b < 2:
                start_x(c, b + 2)
            elif not last:
                start_x(c + 1, b - 2)
            wait_x(b)
            if b == 0:
                wait_wpe(p)
                if not last:
                    start_wpe(c + 1, 1 - p)
            add_step(b, p)
            start_out(c, b)

        start_wpe(0, 0)
        start_x(0, 0)
        start_x(0, 1)
        step(0, 0, 0, first=True)
        step(0, 1, 0, first=True)
        step(0, 2, 0)
        step(0, 3, 0)

        def pair(j, carry):
            c = 2 * j + 1
            for b in range(B):
                step(c, b, 1)
            for b in range(B):
                step(c + 1, b, 0)
            return carry

        lax.fori_loop(0, (NCHUNK - 2) // 2, pair, 0)

        cl = NCHUNK - 1
        for b in range(B):
            step(cl, b, 1, last=True)
        wait_out(2)
        wait_out(3)

    return sc_add


def kernel(x, wpe):
    b, t, d = x.shape
    sc_add = _make_sc_kernel(b, t, d)
    return sc_add(x, wpe)


# speedup vs baseline: 3.5887x; 1.0056x over previous
import functools

import jax
import jax.numpy as jnp
from jax import lax
from jax.experimental import pallas as pl
from jax.experimental.pallas import tpu as pltpu
from jax.experimental.pallas import tpu_sc as plsc

NW = 32
NS = 16
CH = 8
LANES = 16


def _make_sc_kernel(B, T, D):
    TW = T // NW
    NCHUNK = TW // CH

    mesh = plsc.VectorSubcoreMesh(core_axis_name="c", subcore_axis_name="s")

    @functools.partial(
        pl.kernel,
        out_type=jax.ShapeDtypeStruct((B, T, D), jnp.float32),
        mesh=mesh,
        compiler_params=pltpu.CompilerParams(
            use_tc_tiling_on_sc=True,
            disable_bounds_checks=True,
        ),
        scratch_types=[
            pltpu.VMEM((2, CH, D), jnp.float32),
            pltpu.VMEM((B, CH, D), jnp.float32),
            pltpu.VMEM_SHARED((NS, CH, D), jnp.float32),
            pltpu.SemaphoreType.DMA((2,)),
            pltpu.SemaphoreType.DMA((B,)),
            pltpu.SemaphoreType.DMA,
            pltpu.SemaphoreType.DMA,
            pltpu.SemaphoreType.DMA((B,)),
        ],
    )
    def sc_add(x_hbm, wpe_hbm, out_hbm, wpe_buf, x_buf, sp_out,
               wpe_sem, x_sem, sp_sem, o0_sem, out_sem):
        cid = lax.axis_index("c")
        sid = lax.axis_index("s")
        wid = sid * 2 + cid
        t0w = wid * TW

        def start_x(c, b):
            t0 = t0w + c * CH
            pltpu.async_copy(
                x_hbm.at[b, pl.ds(t0, CH)], x_buf.at[b], x_sem.at[b])

        def wait_x(b):
            pltpu.make_async_copy(
                x_hbm.at[b, pl.ds(0, CH)], x_buf.at[b], x_sem.at[b]).wait()

        def start_wpe(c, p):
            t0 = t0w + c * CH
            pltpu.async_copy(
                wpe_hbm.at[pl.ds(t0, CH)], wpe_buf.at[p], wpe_sem.at[p])

        def wait_wpe(p):
            pltpu.make_async_copy(
                wpe_hbm.at[pl.ds(0, CH)], wpe_buf.at[p], wpe_sem.at[p]).wait()

        def start_sp():
            pltpu.async_copy(x_buf.at[0], sp_out.at[sid], sp_sem)

        def wait_sp():
            pltpu.make_async_copy(x_buf.at[0], sp_out.at[sid], sp_sem).wait()

        def start_o0(c):
            t0 = t0w + c * CH
            pltpu.async_copy(
                sp_out.at[sid], out_hbm.at[0, pl.ds(t0, CH)], o0_sem)

        def wait_o0():
            pltpu.make_async_copy(
                sp_out.at[sid], out_hbm.at[0, pl.ds(0, CH)], o0_sem).wait()

        def start_out(c, b):
            t0 = t0w + c * CH
            pltpu.async_copy(
                x_buf.at[b], out_hbm.at[b, pl.ds(t0, CH)], out_sem.at[b])

        def wait_out(b):
            pltpu.make_async_copy(
                x_buf.at[b], out_hbm.at[b, pl.ds(0, CH)],
                out_sem.at[b]).wait()

        def add_step(b, p):
            for r in range(CH):
                @plsc.parallel_loop(0, D // LANES, unroll=8)
                def _(i):
                    sl = pl.ds(i * LANES, LANES)
                    plsc.addupdate(x_buf.at[b, r, sl], wpe_buf[p, r, sl])

        def step(c, b, p, *, first=False, last=False):
            if b == 0 and not first:
                wait_out(2)
            elif b == 1:
                wait_sp()
                start_o0(c)
                if not first:
                    wait_out(3)
            elif b == 3:
                wait_out(1)
            if b < 2:
                start_x(c, b + 2)
            elif not last:
                start_x(c + 1, b - 2)
            wait_x(b)
            if b == 0:
                wait_wpe(p)
                if not last:
                    start_wpe(c + 1, 1 - p)
            add_step(b, p)
            if b == 0:
                if not first:
                    wait_o0()
                start_sp()
            else:
                start_out(c, b)

        start_wpe(0, 0)
        start_x(0, 0)
        start_x(0, 1)
        for b in range(B):
            step(0, b, 0, first=True)

        def pair(j, carry):
            c = 2 * j + 1
            for b in range(B):
                step(c, b, 1)
            for b in range(B):
                step(c + 1, b, 0)
            return carry

        lax.fori_loop(0, (NCHUNK - 2) // 2, pair, 0)

        cl = NCHUNK - 1
        for b in range(B):
            step(cl, b, 1, last=True)
        wait_o0()
        wait_out(2)
        wait_out(3)

    return sc_add


def kernel(x, wpe):
    b, t, d = x.shape
    sc_add = _make_sc_kernel(b, t, d)
    return sc_add(x, wpe)
